# baseline (speedup 1.0000x reference)
import functools

import jax
import jax.numpy as jnp
from jax import lax
from jax.experimental import pallas as pl
from jax.experimental.pallas import tpu as pltpu
from jax.experimental.pallas import tpu_sc as plsc

NU = 4
N = 4096
NQ = 4096
K = 30
H = 64
KP = 32
RB = 64
NB_A = NU * NQ // RB
BPF = NQ // RB
ROWS_B = NU * NQ * KP // 128
RB_B = 256
NB_B = ROWS_B // RB_B
INF = 3.0e38
SENT = 3.0e38
IMAX = 2147483647

NC = 2
NS = 16
LN = 16
NW = NC * NS
GTOT = NU * NQ * KP
GPW = GTOT // NW
TBL = NU * N


def _select_kernel(qx_ref, qy_ref, px_ref, py_ref, idx_ref, qxr_ref, qyr_ref):
    f = pl.program_id(0) // BPF
    qx = qx_ref[0]
    qy = qy_ref[0]
    px = px_ref[0]
    py = py_ref[0]
    q_sq = qx * qx + qy * qy
    p_sq = px * px + py * py
    qxb = qx.astype(jnp.bfloat16).astype(jnp.float32)
    qyb = qy.astype(jnp.bfloat16).astype(jnp.float32)
    pxb = px.astype(jnp.bfloat16).astype(jnp.float32)
    pyb = py.astype(jnp.bfloat16).astype(jnp.float32)
    cross = qxb * pxb + qyb * pyb
    d2 = q_sq - 2.0 * cross + p_sq

    b = lax.bitcast_convert_type(d2, jnp.int32)
    skey = b ^ (lax.shift_right_arithmetic(b, 31) & 0x7FFFFFFF)
    lane_j = lax.broadcasted_iota(jnp.int32, (1, N), 1)
    vrow = lax.shift_right_logical(lane_j, 7)
    ikey = (skey & ~31) | vrow
    fkey_all = lax.bitcast_convert_type(ikey + 0x3D800020, jnp.float32)

    RG = RB // 8
    groups = []
    for g in range(RB // RG):
        work = fkey_all[g * RG:(g + 1) * RG, :]
        liota = lax.broadcasted_iota(
            jnp.int32, (RG, 128), 1).astype(jnp.float32)
        cols = []
        for _ in range(K):
            parts = [work[:, c * 128:(c + 1) * 128] for c in range(N // 128)]
            while len(parts) > 1:
                parts = [jnp.minimum(parts[i], parts[i + 1])
                         for i in range(0, len(parts), 2)]
            colmin = parts[0]
            m = jnp.min(colmin, axis=1, keepdims=True)
            oc = colmin == m
            l = jnp.min(jnp.where(oc, liota, 128.0), axis=1, keepdims=True)
            li = l.astype(jnp.int32)
            v = lax.bitcast_convert_type(m, jnp.int32) & 31
            jloc = v * 128 + li
            cols.append(jloc + f * N)
            work = jnp.where(lane_j == jloc, SENT, work)
        z = jnp.zeros((RG, KP - K), jnp.int32)
        groups.append(jnp.concatenate(cols + [z], axis=1))
    idx_ref[0] = jnp.concatenate(groups, axis=0)
    qxr_ref[0] = jnp.broadcast_to(qx, (RB, KP))
    qyr_ref[0] = jnp.broadcast_to(qy, (RB, KP))


def _run_select(u, init_x, init_y, x, y):
    px = init_x.reshape(NU, 1, N)
    py = init_y.reshape(NU, 1, N)
    qxA = x.reshape(NB_A, RB, 1)
    qyA = y.reshape(NB_A, RB, 1)
    return pl.pallas_call(
        _select_kernel,
        grid=(NB_A,),
        in_specs=[
            pl.BlockSpec((1, RB, 1), lambda i: (i, 0, 0)),
            pl.BlockSpec((1, RB, 1), lambda i: (i, 0, 0)),
            pl.BlockSpec((1, 1, N), lambda i: (i // BPF, 0, 0)),
            pl.BlockSpec((1, 1, N), lambda i: (i // BPF, 0, 0)),
        ],
        out_specs=[
            pl.BlockSpec((1, RB, KP), lambda i: (i, 0, 0)),
            pl.BlockSpec((1, RB, KP), lambda i: (i, 0, 0)),
            pl.BlockSpec((1, RB, KP), lambda i: (i, 0, 0)),
        ],
        out_shape=[
            jax.ShapeDtypeStruct((NB_A, RB, KP), jnp.int32),
            jax.ShapeDtypeStruct((NB_A, RB, KP), jnp.float32),
            jax.ShapeDtypeStruct((NB_A, RB, KP), jnp.float32),
        ],
    )(qxA, qyA, px, py)


def _gather_sc(idx_hbm, tx_hbm, ty_hbm, tl_hbm, ox_hbm, oy_hbm, ol_hbm,
               idx_v, tx_v, ty_v, tl_v, ox_v, oy_v, ol_v):
    wid = lax.axis_index("s") * NC + lax.axis_index("c")
    base = wid * GPW
    pltpu.sync_copy(tx_hbm, tx_v)
    pltpu.sync_copy(ty_hbm, ty_v)
    pltpu.sync_copy(tl_hbm, tl_v)
    pltpu.sync_copy(idx_hbm.at[pl.ds(base, GPW)], idx_v)

    def body(i, carry):
        sl = pl.ds(i * LN, LN)
        iv = idx_v[sl]
        ox_v[sl] = plsc.load_gather(tx_v, [iv])
        oy_v[sl] = plsc.load_gather(ty_v, [iv])
        ol_v[sl] = plsc.load_gather(tl_v, [iv])
        return carry

    lax.fori_loop(0, GPW // LN, body, 0)
    pltpu.sync_copy(ox_v, ox_hbm.at[pl.ds(base, GPW)])
    pltpu.sync_copy(oy_v, oy_hbm.at[pl.ds(base, GPW)])
    pltpu.sync_copy(ol_v, ol_hbm.at[pl.ds(base, GPW)])


def _run_gather(idx_flat, tx, ty, tl):
    mesh = plsc.VectorSubcoreMesh(core_axis_name="c", subcore_axis_name="s")
    fo = jax.ShapeDtypeStruct((GTOT,), jnp.float32)
    fn = functools.partial(
        pl.kernel,
        mesh=mesh,
        compiler_params=pltpu.CompilerParams(needs_layout_passes=False),
        out_type=[fo, fo, fo],
        scratch_types=[
            pltpu.VMEM((GPW,), jnp.int32),
            pltpu.VMEM((TBL,), jnp.float32),
            pltpu.VMEM((TBL,), jnp.float32),
            pltpu.VMEM((TBL,), jnp.float32),
            pltpu.VMEM((GPW,), jnp.float32),
            pltpu.VMEM((GPW,), jnp.float32),
            pltpu.VMEM((GPW,), jnp.float32),
        ],
    )(_gather_sc)
    return fn(idx_flat, tx, ty, tl)


def _mlp_kernel(qx_ref, qy_ref, nx_ref, ny_ref, nl_ref, w1_ref, b1_ref, w2_ref,
                out_ref):
    qx = qx_ref[...]
    qy = qy_ref[...]
    nx = nx_ref[...]
    ny = ny_ref[...]
    nl = nl_ref[...]
    relx = nx - qx
    rely = ny - qy
    s = jnp.zeros_like(nx)
    for hh in range(H):
        pre = (relx * w1_ref[0, hh, :] + rely * w1_ref[1, hh, :]
               + nx * w1_ref[2, hh, :] + ny * w1_ref[3, hh, :] + b1_ref[hh, :])
        s = s + jnp.tanh(pre) * w2_ref[hh, :]
    lane = lax.broadcasted_iota(jnp.int32, s.shape, 1)
    kmask = (lane & (KP - 1)) < K
    e = jnp.where(kmask, jnp.exp(s), 0.0)
    segr = lax.broadcasted_iota(jnp.int32, (128, 128), 0) // KP
    segc = lax.broadcasted_iota(jnp.int32, (128, 128), 1) // KP
    segmat = (segr == segc).astype(jnp.float32)
    den = jnp.dot(e, segmat, preferred_element_type=jnp.float32)
    num = jnp.dot(e * nl, segmat, preferred_element_type=jnp.float32)
    out_ref[...] = num / den


def _run_mlp(qx4, qy4, nx4, ny4, nl4, W1, b1, W2):
    w1bc = jnp.broadcast_to(W1.T.reshape(H, 4, 1), (H, 4, 128))
    w1bc = jnp.transpose(w1bc, (1, 0, 2))
    b1bc = jnp.broadcast_to(b1.reshape(H, 1), (H, 128))
    w2bc = jnp.broadcast_to(W2.reshape(H, 1), (H, 128))
    return pl.pallas_call(
        _mlp_kernel,
        grid=(NB_B,),
        in_specs=[
            pl.BlockSpec((RB_B, 128), lambda i: (i, 0)),
            pl.BlockSpec((RB_B, 128), lambda i: (i, 0)),
            pl.BlockSpec((RB_B, 128), lambda i: (i, 0)),
            pl.BlockSpec((RB_B, 128), lambda i: (i, 0)),
            pl.BlockSpec((RB_B, 128), lambda i: (i, 0)),
            pl.BlockSpec((4, H, 128), lambda i: (0, 0, 0)),
            pl.BlockSpec((H, 128), lambda i: (0, 0)),
            pl.BlockSpec((H, 128), lambda i: (0, 0)),
        ],
        out_specs=pl.BlockSpec((RB_B, 128), lambda i: (i, 0)),
        out_shape=jax.ShapeDtypeStruct((ROWS_B, 128), jnp.float32),
    )(qx4, qy4, nx4, ny4, nl4, w1bc, b1bc, w2bc)


def kernel(u, init_x, init_y, x, y, W1, b1, W2, b2):
    idx, qxr, qyr = _run_select(u, init_x, init_y, x, y)
    gx, gy, gl = _run_gather(idx.reshape(-1),
                             init_x.reshape(-1), init_y.reshape(-1),
                             u.reshape(-1))
    out4 = _run_mlp(qxr.reshape(ROWS_B, 128),
                    qyr.reshape(ROWS_B, 128),
                    gx.reshape(ROWS_B, 128),
                    gy.reshape(ROWS_B, 128),
                    gl.reshape(ROWS_B, 128),
                    W1, b1, W2)
    return out4.reshape(NU * NQ, KP)[:, 0]


# speedup vs baseline: 1.2386x; 1.2386x over previous
import functools

import jax
import jax.numpy as jnp
from jax import lax
from jax.experimental import pallas as pl
from jax.experimental.pallas import tpu as pltpu
from jax.experimental.pallas import tpu_sc as plsc

NU = 4
N = 4096
NQ = 4096
K = 30
H = 64
KP = 32
RB = 128
NB_A = NU * NQ // RB
BPF = NQ // RB
ROWS_B = NU * NQ * KP // 128
RB_B = 256
NB_B = ROWS_B // RB_B
INF = 3.0e38
SENT = 3.0e38
IMAX = 2147483647

NC = 2
NS = 16
LN = 16
NW = NC * NS
GTOT = NU * NQ * KP
GPW = GTOT // NW
TBL = NU * N


def _select_kernel(qx_ref, qy_ref, px_ref, py_ref, idx_ref):
    f = pl.program_id(0) // BPF
    qx = qx_ref[0]
    qy = qy_ref[0]
    px = px_ref[0]
    py = py_ref[0]
    q_sq = qx * qx + qy * qy
    p_sq = px * px + py * py
    qxb = qx.astype(jnp.bfloat16).astype(jnp.float32)
    qyb = qy.astype(jnp.bfloat16).astype(jnp.float32)
    pxb = px.astype(jnp.bfloat16).astype(jnp.float32)
    pyb = py.astype(jnp.bfloat16).astype(jnp.float32)
    cross = qxb * pxb + qyb * pyb
    d2 = q_sq - 2.0 * cross + p_sq

    b = lax.bitcast_convert_type(d2, jnp.int32)
    skey = b ^ (lax.shift_right_arithmetic(b, 31) & 0x7FFFFFFF)
    lane_j = lax.broadcasted_iota(jnp.int32, (1, N), 1)
    vrow = lax.shift_right_logical(lane_j, 7)
    ikey = (skey & ~31) | vrow
    fkey_all = lax.bitcast_convert_type(ikey + 0x3D800020, jnp.float32)

    RG = RB // 8
    groups = []
    for g in range(RB // RG):
        work = fkey_all[g * RG:(g + 1) * RG, :]
        liota = lax.broadcasted_iota(
            jnp.int32, (RG, 128), 1).astype(jnp.float32)
        cols = []
        for _ in range(K):
            parts = [work[:, c * 128:(c + 1) * 128] for c in range(N // 128)]
            while len(parts) > 1:
                parts = [jnp.minimum(parts[i], parts[i + 1])
                         for i in range(0, len(parts), 2)]
            colmin = parts[0]
            m = jnp.min(colmin, axis=1, keepdims=True)
            oc = colmin == m
            l = jnp.min(jnp.where(oc, liota, 128.0), axis=1, keepdims=True)
            li = l.astype(jnp.int32)
            v = lax.bitcast_convert_type(m, jnp.int32) & 31
            jloc = v * 128 + li
            cols.append(jloc + f * N)
            work = jnp.where(lane_j == jloc, SENT, work)
        z = jnp.zeros((RG, KP - K), jnp.int32)
        groups.append(jnp.concatenate(cols + [z], axis=1))
    idx_ref[0] = jnp.concatenate(groups, axis=0)


def _run_select(u, init_x, init_y, x, y):
    px = init_x.reshape(NU, 1, N)
    py = init_y.reshape(NU, 1, N)
    qxA = x.reshape(NB_A, RB, 1)
    qyA = y.reshape(NB_A, RB, 1)
    return pl.pallas_call(
        _select_kernel,
        grid=(NB_A,),
        in_specs=[
            pl.BlockSpec((1, RB, 1), lambda i: (i, 0, 0)),
            pl.BlockSpec((1, RB, 1), lambda i: (i, 0, 0)),
            pl.BlockSpec((1, 1, N), lambda i: (i // BPF, 0, 0)),
            pl.BlockSpec((1, 1, N), lambda i: (i // BPF, 0, 0)),
        ],
        out_specs=pl.BlockSpec((1, RB, KP), lambda i: (i, 0, 0)),
        out_shape=jax.ShapeDtypeStruct((NB_A, RB, KP), jnp.int32),
    )(qxA, qyA, px, py)


def _gather_sc(idx_hbm, tx_hbm, ty_hbm, tl_hbm, ox_hbm, oy_hbm, ol_hbm,
               idx_v, tx_v, ty_v, tl_v, ox_v, oy_v, ol_v):
    wid = lax.axis_index("s") * NC + lax.axis_index("c")
    base = wid * GPW
    pltpu.sync_copy(tx_hbm, tx_v)
    pltpu.sync_copy(ty_hbm, ty_v)
    pltpu.sync_copy(tl_hbm, tl_v)
    pltpu.sync_copy(idx_hbm.at[pl.ds(base, GPW)], idx_v)

    def body(i, carry):
        sl = pl.ds(i * LN, LN)
        iv = idx_v[sl]
        ox_v[sl] = plsc.load_gather(tx_v, [iv])
        oy_v[sl] = plsc.load_gather(ty_v, [iv])
        ol_v[sl] = plsc.load_gather(tl_v, [iv])
        return carry

    lax.fori_loop(0, GPW // LN, body, 0)
    pltpu.sync_copy(ox_v, ox_hbm.at[pl.ds(base, GPW)])
    pltpu.sync_copy(oy_v, oy_hbm.at[pl.ds(base, GPW)])
    pltpu.sync_copy(ol_v, ol_hbm.at[pl.ds(base, GPW)])


def _run_gather(idx_flat, tx, ty, tl):
    mesh = plsc.VectorSubcoreMesh(core_axis_name="c", subcore_axis_name="s")
    fo = jax.ShapeDtypeStruct((GTOT,), jnp.float32)
    fn = functools.partial(
        pl.kernel,
        mesh=mesh,
        compiler_params=pltpu.CompilerParams(needs_layout_passes=False),
        out_type=[fo, fo, fo],
        scratch_types=[
            pltpu.VMEM((GPW,), jnp.int32),
            pltpu.VMEM((TBL,), jnp.float32),
            pltpu.VMEM((TBL,), jnp.float32),
            pltpu.VMEM((TBL,), jnp.float32),
            pltpu.VMEM((GPW,), jnp.float32),
            pltpu.VMEM((GPW,), jnp.float32),
            pltpu.VMEM((GPW,), jnp.float32),
        ],
    )(_gather_sc)
    return fn(idx_flat, tx, ty, tl)


def _mlp_kernel(qx_ref, qy_ref, nx_ref, ny_ref, nl_ref, w1_ref, b1_ref, w2_ref,
                out_ref):
    qx = qx_ref[...]
    qy = qy_ref[...]
    nx = nx_ref[...]
    ny = ny_ref[...]
    nl = nl_ref[...]
    relx = nx - qx
    rely = ny - qy
    s = jnp.zeros_like(nx)
    for hh in range(H):
        pre = (relx * w1_ref[0, hh, :] + rely * w1_ref[1, hh, :]
               + nx * w1_ref[2, hh, :] + ny * w1_ref[3, hh, :] + b1_ref[hh, :])
        s = s + jnp.tanh(pre) * w2_ref[hh, :]
    lane = lax.broadcasted_iota(jnp.int32, s.shape, 1)
    kmask = (lane & (KP - 1)) < K
    e = jnp.where(kmask, jnp.exp(s), 0.0)
    segr = lax.broadcasted_iota(jnp.int32, (128, 128), 0) // KP
    segc = lax.broadcasted_iota(jnp.int32, (128, 128), 1) // KP
    segmat = (segr == segc).astype(jnp.float32)
    den = jnp.dot(e, segmat, preferred_element_type=jnp.float32)
    num = jnp.dot(e * nl, segmat, preferred_element_type=jnp.float32)
    out_ref[...] = num / den


def _run_mlp(x, y, nx4, ny4, nl4, W1, b1, W2):
    xq = x.reshape(-1)
    yq = y.reshape(-1)
    qx4 = jnp.broadcast_to(xq[:, None], (NU * NQ, KP)).reshape(ROWS_B, 128)
    qy4 = jnp.broadcast_to(yq[:, None], (NU * NQ, KP)).reshape(ROWS_B, 128)
    w1bc = jnp.broadcast_to(W1.T.reshape(H, 4, 1), (H, 4, 128))
    w1bc = jnp.transpose(w1bc, (1, 0, 2))
    b1bc = jnp.broadcast_to(b1.reshape(H, 1), (H, 128))
    w2bc = jnp.broadcast_to(W2.reshape(H, 1), (H, 128))
    return pl.pallas_call(
        _mlp_kernel,
        grid=(NB_B,),
        in_specs=[
            pl.BlockSpec((RB_B, 128), lambda i: (i, 0)),
            pl.BlockSpec((RB_B, 128), lambda i: (i, 0)),
            pl.BlockSpec((RB_B, 128), lambda i: (i, 0)),
            pl.BlockSpec((RB_B, 128), lambda i: (i, 0)),
            pl.BlockSpec((RB_B, 128), lambda i: (i, 0)),
            pl.BlockSpec((4, H, 128), lambda i: (0, 0, 0)),
            pl.BlockSpec((H, 128), lambda i: (0, 0)),
            pl.BlockSpec((H, 128), lambda i: (0, 0)),
        ],
        out_specs=pl.BlockSpec((RB_B, 128), lambda i: (i, 0)),
        out_shape=jax.ShapeDtypeStruct((ROWS_B, 128), jnp.float32),
    )(qx4, qy4, nx4, ny4, nl4, w1bc, b1bc, w2bc)


def kernel(u, init_x, init_y, x, y, W1, b1, W2, b2):
    idx = _run_select(u, init_x, init_y, x, y)
    gx, gy, gl = _run_gather(idx.reshape(-1),
                             init_x.reshape(-1), init_y.reshape(-1),
                             u.reshape(-1))
    out4 = _run_mlp(x, y,
                    gx.reshape(ROWS_B, 128),
                    gy.reshape(ROWS_B, 128),
                    gl.reshape(ROWS_B, 128),
                    W1, b1, W2)
    return out4.reshape(NU * NQ, KP)[:, 0]
